# baseline (speedup 1.0000x reference)
import functools

import jax
import jax.numpy as jnp
from jax.experimental import pallas as pl

_MIN_K = 16
_EPS = 1e-10


def _soft_knn_block(fr_ref, fa_ref, out_ref, *, n_bisect):
    fr = fr_ref[...]
    fa = fa_ref[...]
    r = fr.shape[0]
    n = fa.shape[0]

    sqr = jnp.sum(fr * fr, axis=1, keepdims=True)
    sqa = jnp.sum(fa * fa, axis=1, keepdims=True)
    ones_n = jnp.ones_like(sqa)
    dot = jax.lax.dot_general(
        fr * -2.0, fa, (((1,), (1,)), ((), ())),
        preferred_element_type=jnp.float32)
    d2 = jnp.maximum(sqr + sqa.reshape(1, n) + dot, 0.0)

    slices = [d2[:, j * 128:(j + 1) * 128] for j in range(n // 128)]
    while len(slices) > 1:
        half = len(slices) // 2
        slices = [jnp.minimum(slices[i], slices[i + half])
                  for i in range(half)] + slices[2 * half:]
    c = slices[0]

    kk = jnp.float32(_MIN_K + 1)

    out_ref[...] = d2 * jax.lax.rsqrt(jnp.maximum(d2, 1e-37))

    lo_c = jnp.zeros((r, 1), jnp.float32)
    hi_c = jnp.max(c, axis=1, keepdims=True)
    for _ in range(10):
        mid = 0.5 * (lo_c + hi_c)
        cnt = jnp.sum(jnp.where(c <= mid, 1.0, 0.0), axis=1, keepdims=True)
        ge = cnt >= kk
        lo_c = jnp.where(ge, lo_c, mid)
        hi_c = jnp.where(ge, mid, hi_c)
    ub = hi_c

    lo = jnp.zeros_like(ub)
    hi = ub
    for _ in range(n_bisect):
        mid = 0.5 * (lo + hi)
        mask = jnp.where(d2 <= mid, 1.0, 0.0)
        cnt = jax.lax.dot_general(mask, ones_n, (((1,), (0,)), ((), ())),
                                  preferred_element_type=jnp.float32)
        ge = cnt >= kk
        lo = jnp.where(ge, lo, mid)
        hi = jnp.where(ge, mid, hi)

    thrp = jnp.sqrt(0.5 * (lo + hi)) + _EPS
    dist = out_ref[...]
    w = jnp.maximum(thrp - dist, 0.0)
    norm = jax.lax.dot_general(w, ones_n, (((1,), (0,)), ((), ())),
                               preferred_element_type=jnp.float32)
    out_ref[...] = w * (1.0 / jnp.maximum(norm, 1e-12))


def kernel(features):
    n, d = features.shape
    block_r = 512
    grid = (n // block_r,)
    return pl.pallas_call(
        functools.partial(_soft_knn_block, n_bisect=9),
        grid=grid,
        in_specs=[
            pl.BlockSpec((block_r, d), lambda i: (i, 0)),
            pl.BlockSpec((n, d), lambda i: (0, 0)),
        ],
        out_specs=pl.BlockSpec((block_r, n), lambda i: (i, 0)),
        out_shape=jax.ShapeDtypeStruct((n, n), jnp.float32),
    )(features, features)


# speedup vs baseline: 1.9176x; 1.9176x over previous
import functools

import jax
import jax.numpy as jnp
from jax.experimental import pallas as pl

_MIN_K = 16
_EPS = 1e-10


def _soft_knn_block(fr_ref, fa_ref, out_ref, *, n_bisect):
    fr = fr_ref[...]
    fa = fa_ref[...]
    n = fa.shape[0]

    sqr = jnp.sum(fr * fr, axis=1, keepdims=True)
    sqa = jnp.sum(fa * fa, axis=1, keepdims=True)
    dot = jax.lax.dot_general(
        fr * -2.0, fa, (((1,), (1,)), ((), ())),
        preferred_element_type=jnp.float32)
    d2 = jnp.maximum(sqr + sqa.reshape(1, n) + dot, 0.0)

    slices = [d2[:, j * 128:(j + 1) * 128] for j in range(n // 128)]
    while len(slices) > 1:
        half = len(slices) // 2
        slices = [jnp.minimum(slices[i], slices[i + half])
                  for i in range(half)] + slices[2 * half:]
    c = slices[0]

    kk = jnp.float32(_MIN_K + 1)

    def _bisect_c(_, carry):
        lo, hi = carry
        cnt = jnp.sum(jnp.where(c <= 0.5 * (lo + hi), 1.0, 0.0),
                      axis=1, keepdims=True)
        ge = cnt >= kk
        mid = 0.5 * (lo + hi)
        return jnp.where(ge, lo, mid), jnp.where(ge, mid, hi)

    cmax = jnp.max(c, axis=1, keepdims=True)
    _, ub = jax.lax.fori_loop(0, 10, _bisect_c, (jnp.zeros_like(cmax), cmax))


    def _bisect(_, carry):
        lo, hi = carry
        mid = 0.5 * (lo + hi)
        cnt = jnp.sum(jnp.where(d2 <= mid, 1.0, 0.0), axis=1, keepdims=True)
        ge = cnt >= kk
        return jnp.where(ge, lo, mid), jnp.where(ge, mid, hi)

    lo0 = jnp.zeros_like(ub)
    lo, hi = jax.lax.fori_loop(0, n_bisect, _bisect, (lo0, ub))

    thr = jnp.sqrt(0.5 * (lo + hi))
    dist = d2 * jax.lax.rsqrt(jnp.maximum(d2, 1e-37))
    w = jnp.maximum(thr - dist + _EPS, 0.0)
    norm = jnp.sum(w, axis=1, keepdims=True)
    out_ref[...] = w / jnp.maximum(norm, 1e-12)


def kernel(features):
    n, d = features.shape
    block_r = 512
    grid = (n // block_r,)
    return pl.pallas_call(
        functools.partial(_soft_knn_block, n_bisect=8),
        grid=grid,
        in_specs=[
            pl.BlockSpec((block_r, d), lambda i: (i, 0)),
            pl.BlockSpec((n, d), lambda i: (0, 0)),
        ],
        out_specs=pl.BlockSpec((block_r, n), lambda i: (i, 0)),
        out_shape=jax.ShapeDtypeStruct((n, n), jnp.float32),
    )(features, features)


# speedup vs baseline: 1.9790x; 1.0320x over previous
import functools

import jax
import jax.numpy as jnp
from jax.experimental import pallas as pl

_MIN_K = 16
_EPS = 1e-10


def _soft_knn_block(fr_ref, fa_ref, out_ref, *, n_bisect):
    fr = fr_ref[...]
    fa = fa_ref[...]
    n = fa.shape[0]

    sqr = jnp.sum(fr * fr, axis=1, keepdims=True)
    sqa = jnp.sum(fa * fa, axis=1, keepdims=True)
    dot = jax.lax.dot_general(
        fr * -2.0, fa, (((1,), (1,)), ((), ())),
        preferred_element_type=jnp.float32)
    d2 = jnp.maximum(sqr + sqa.reshape(1, n) + dot, 0.0)

    slices = [d2[:, j * 128:(j + 1) * 128] for j in range(n // 128)]
    while len(slices) > 1:
        half = len(slices) // 2
        slices = [jnp.minimum(slices[i], slices[i + half])
                  for i in range(half)] + slices[2 * half:]
    c = slices[0]

    kk = jnp.float32(_MIN_K + 1)

    def _bisect_c(_, carry):
        lo, hi = carry
        cnt = jnp.sum(jnp.where(c <= 0.5 * (lo + hi), 1.0, 0.0),
                      axis=1, keepdims=True)
        ge = cnt >= kk
        mid = 0.5 * (lo + hi)
        return jnp.where(ge, lo, mid), jnp.where(ge, mid, hi)

    cmax = jnp.max(c, axis=1, keepdims=True)
    _, ub = jax.lax.fori_loop(0, 10, _bisect_c, (jnp.zeros_like(cmax), cmax))


    def _bisect(_, carry):
        lo, hi = carry
        mid = 0.5 * (lo + hi)
        cnt = jnp.sum(jnp.where(d2 <= mid, 1.0, 0.0), axis=1, keepdims=True)
        ge = cnt >= kk
        return jnp.where(ge, lo, mid), jnp.where(ge, mid, hi)

    lo0 = jnp.zeros_like(ub)
    lo, hi = jax.lax.fori_loop(0, n_bisect, _bisect, (lo0, ub))

    thrp = jnp.sqrt(0.5 * (lo + hi)) + _EPS
    dist = d2 * jax.lax.rsqrt(jnp.maximum(d2, 1e-37))
    w = jnp.maximum(thrp - dist, 0.0)
    norm = jnp.sum(w, axis=1, keepdims=True)
    out_ref[...] = w * (1.0 / jnp.maximum(norm, 1e-12))


def kernel(features):
    n, d = features.shape
    block_r = 512
    grid = (n // block_r,)
    return pl.pallas_call(
        functools.partial(_soft_knn_block, n_bisect=8),
        grid=grid,
        in_specs=[
            pl.BlockSpec((block_r, d), lambda i: (i, 0)),
            pl.BlockSpec((n, d), lambda i: (0, 0)),
        ],
        out_specs=pl.BlockSpec((block_r, n), lambda i: (i, 0)),
        out_shape=jax.ShapeDtypeStruct((n, n), jnp.float32),
    )(features, features)
